# baseline (speedup 1.0000x reference)
import functools

import jax
import jax.numpy as jnp
from jax import lax
from jax.experimental import pallas as pl
from jax.experimental.pallas import tpu as pltpu
from jax.experimental.pallas import tpu_sc as plsc

N = 10000
D = 128
E = 320000

NC = 2
NS = 16
NW = NC * NS
LANES = 16

NPAD = 10112
DUMMY = N
CH = 128
NCH = 80
EPT = NCH * CH
EPAD = NW * EPT
ZROWS = NPAD // NS

_mesh = plsc.VectorSubcoreMesh(
    core_axis_name="c", subcore_axis_name="s", num_cores=NC, num_subcores=NS)
_sc_params = pltpu.CompilerParams(needs_layout_passes=False)


@functools.partial(
    pl.kernel,
    out_type=jax.ShapeDtypeStruct((NW, NPAD), jnp.float32),
    mesh=_mesh,
    scratch_types=[
        pltpu.VMEM((EPT,), jnp.int32),
        pltpu.VMEM((NPAD,), jnp.float32),
    ],
    compiler_params=_sc_params,
)
def _deg_kernel(dst_hbm, out_hbm, dst_v, hist_v):
    c = lax.axis_index("c")
    s = lax.axis_index("s")
    wid = s * NC + c
    pltpu.sync_copy(dst_hbm.at[wid], dst_v)
    zeros16 = jnp.zeros((LANES,), jnp.float32)
    ones16 = jnp.ones((LANES,), jnp.float32)

    def zbody(i, carry):
        hist_v[pl.ds(i * LANES, LANES)] = zeros16
        return carry

    lax.fori_loop(0, NPAD // LANES, zbody, 0)

    def ebody(i, carry):
        idx = dst_v[pl.ds(i * LANES, LANES)]
        plsc.addupdate_scatter(hist_v, [idx], ones16)
        return carry

    lax.fori_loop(0, EPT // LANES, ebody, 0)
    pltpu.sync_copy(hist_v, out_hbm.at[wid])


@functools.partial(
    pl.kernel,
    out_type=jax.ShapeDtypeStruct((NC, NPAD, D), jnp.float32),
    mesh=_mesh,
    scratch_types=[
        pltpu.VMEM((NCH, CH), jnp.int32),
        pltpu.VMEM((1, CH), jnp.int32),
        pltpu.VMEM((1, CH), jnp.int32),
        pltpu.VMEM((1, CH, D), jnp.float32),
        pltpu.VMEM_SHARED((NPAD, D), jnp.float32),
    ],
    compiler_params=_sc_params,
)
def _prop_kernel(table_hbm, packed_hbm, out_hbm,
                 packed_v, sidx, didx, rows_v, accum):
    c = lax.axis_index("c")
    s = lax.axis_index("s")
    wid = s * NC + c
    pltpu.sync_copy(packed_hbm.at[wid], packed_v)

    def unpack(jn):
        for i in range(CH // LANES):
            v = packed_v[jn, pl.ds(i * LANES, LANES)]
            sidx[0, pl.ds(i * LANES, LANES)] = v & 0xFFFF
            didx[0, pl.ds(i * LANES, LANES)] = lax.shift_right_logical(v, 16)

    zeros16 = jnp.zeros((LANES,), jnp.float32)

    def zbody(i, carry):
        rows_v[0, i // (D // LANES), pl.ds((i % (D // LANES)) * LANES, LANES)] = zeros16
        return carry

    lax.fori_loop(0, CH * (D // LANES), zbody, 0)
    for z in range(ZROWS // CH):
        pltpu.sync_copy(rows_v.at[0], accum.at[pl.ds(s * ZROWS + z * CH, CH)])
    rem = ZROWS % CH
    if rem:
        pltpu.sync_copy(rows_v.at[0].at[pl.ds(0, rem)],
                        accum.at[pl.ds(s * ZROWS + (ZROWS // CH) * CH, rem)])
    plsc.subcore_barrier()

    def ebody(j, carry):
        unpack(j)
        pltpu.sync_copy(table_hbm.at[sidx.at[0]], rows_v.at[0])
        pltpu.sync_copy(rows_v.at[0], accum.at[didx.at[0]], add=True)
        return carry

    lax.fori_loop(0, NCH, ebody, 0)
    plsc.subcore_barrier()
    pltpu.sync_copy(accum.at[pl.ds(s * ZROWS, ZROWS)],
                    out_hbm.at[c].at[pl.ds(s * ZROWS, ZROWS)])


def _diag_scale(dis2, mat):
    ri = lax.broadcasted_iota(jnp.int32, (D, D), 0)
    ci = lax.broadcasted_iota(jnp.int32, (D, D), 1)
    diag = jnp.where(ri == ci, jnp.broadcast_to(dis2, (D, D)), 0.0)
    return jnp.dot(diag, mat, preferred_element_type=jnp.float32)


def _scale1_body(x_ref, pd_ref, w_ref, h_ref, dis_ref):
    pd = pd_ref[...][0]
    deg = jnp.sum(pd, axis=0, keepdims=True)
    dis = jnp.where(deg > 0.0, lax.rsqrt(jnp.maximum(deg, 1.0)), 0.0)
    xs = _diag_scale(dis, x_ref[...])
    h_ref[...] = jnp.dot(xs, w_ref[...], preferred_element_type=jnp.float32)
    dis_ref[...] = dis.reshape(1, 1, D)


def _scale2_body(p_ref, dis_ref, w_ref, b_ref, h_ref):
    p = p_ref[...]
    psum = p[0] + p[1]
    dis = dis_ref[...].reshape(1, D)
    h1 = _diag_scale(dis, psum) + b_ref[...]
    h_ref[...] = jnp.dot(_diag_scale(dis, h1), w_ref[...],
                         preferred_element_type=jnp.float32)


def _final_body(p_ref, dis_ref, b_ref, out_ref):
    p = p_ref[...]
    psum = p[0] + p[1]
    dis = dis_ref[...].reshape(1, D)
    out_ref[...] = _diag_scale(dis, psum) + b_ref[...]


BLK = 128
GRID = NPAD // BLK


def kernel(x, edge_index, W1, b1, W2, b2):
    src = edge_index[0]
    dst = edge_index[1]
    pad = EPAD - E
    src_p = jnp.concatenate([src, jnp.zeros((pad,), jnp.int32)])
    dst_p = jnp.concatenate([dst, jnp.full((pad,), DUMMY, jnp.int32)])
    packed3 = (src_p | (dst_p << 16)).reshape(NW, NCH, CH)
    dst2 = dst_p.reshape(NW, EPT)
    x_pad = jnp.pad(x, ((0, NPAD - N), (0, 0)))
    b1r = b1.reshape(1, D)
    b2r = b2.reshape(1, D)

    pd = _deg_kernel(dst2)
    pd3 = pd.reshape(NW, GRID, BLK).transpose(1, 0, 2)

    h1s, dis3 = pl.pallas_call(
        _scale1_body,
        grid=(GRID,),
        in_specs=[
            pl.BlockSpec((BLK, D), lambda i: (i, 0)),
            pl.BlockSpec((1, NW, BLK), lambda i: (i, 0, 0)),
            pl.BlockSpec((D, D), lambda i: (0, 0)),
        ],
        out_specs=[
            pl.BlockSpec((BLK, D), lambda i: (i, 0)),
            pl.BlockSpec((1, 1, BLK), lambda i: (i, 0, 0)),
        ],
        out_shape=[
            jax.ShapeDtypeStruct((NPAD, D), jnp.float32),
            jax.ShapeDtypeStruct((GRID, 1, BLK), jnp.float32),
        ],
    )(x_pad, pd3, W1)

    p1 = _prop_kernel(h1s, packed3)

    h2s = pl.pallas_call(
        _scale2_body,
        grid=(GRID,),
        in_specs=[
            pl.BlockSpec((NC, BLK, D), lambda i: (0, i, 0)),
            pl.BlockSpec((1, 1, BLK), lambda i: (i, 0, 0)),
            pl.BlockSpec((D, D), lambda i: (0, 0)),
            pl.BlockSpec((1, D), lambda i: (0, 0)),
        ],
        out_specs=pl.BlockSpec((BLK, D), lambda i: (i, 0)),
        out_shape=jax.ShapeDtypeStruct((NPAD, D), jnp.float32),
    )(p1, dis3, W2, b1r)

    p2 = _prop_kernel(h2s, packed3)

    out = pl.pallas_call(
        _final_body,
        grid=(GRID,),
        in_specs=[
            pl.BlockSpec((NC, BLK, D), lambda i: (0, i, 0)),
            pl.BlockSpec((1, 1, BLK), lambda i: (i, 0, 0)),
            pl.BlockSpec((1, D), lambda i: (0, 0)),
        ],
        out_specs=pl.BlockSpec((BLK, D), lambda i: (i, 0)),
        out_shape=jax.ShapeDtypeStruct((NPAD, D), jnp.float32),
    )(p2, dis3, b2r)

    return out[:N]


# speedup vs baseline: 1.0173x; 1.0173x over previous
import functools

import jax
import jax.numpy as jnp
from jax import lax
from jax.experimental import pallas as pl
from jax.experimental.pallas import tpu as pltpu
from jax.experimental.pallas import tpu_sc as plsc

N = 10000
D = 128
E = 320000

NC = 2
NS = 16
NW = NC * NS
LANES = 16

NPAD = 10112
DUMMY = N
CH = 128
NCH = 80
EPT = NCH * CH
EPAD = NW * EPT
ZROWS = NPAD // NS

_mesh = plsc.VectorSubcoreMesh(
    core_axis_name="c", subcore_axis_name="s", num_cores=NC, num_subcores=NS)
_sc_params = pltpu.CompilerParams(needs_layout_passes=False)


@functools.partial(
    pl.kernel,
    out_type=jax.ShapeDtypeStruct((NW, NPAD), jnp.float32),
    mesh=_mesh,
    scratch_types=[
        pltpu.VMEM((EPT,), jnp.int32),
        pltpu.VMEM((NPAD,), jnp.float32),
    ],
    compiler_params=_sc_params,
)
def _deg_kernel(dst_hbm, out_hbm, dst_v, hist_v):
    c = lax.axis_index("c")
    s = lax.axis_index("s")
    wid = s * NC + c
    pltpu.sync_copy(dst_hbm.at[wid], dst_v)
    zeros16 = jnp.zeros((LANES,), jnp.float32)
    ones16 = jnp.ones((LANES,), jnp.float32)

    def zbody(i, carry):
        hist_v[pl.ds(i * LANES, LANES)] = zeros16
        return carry

    lax.fori_loop(0, NPAD // LANES, zbody, 0)

    def ebody(i, carry):
        idx = dst_v[pl.ds(i * LANES, LANES)]
        plsc.addupdate_scatter(hist_v, [idx], ones16)
        return carry

    lax.fori_loop(0, EPT // LANES, ebody, 0)
    pltpu.sync_copy(hist_v, out_hbm.at[wid])


@functools.partial(
    pl.kernel,
    out_type=jax.ShapeDtypeStruct((NC, NPAD, D), jnp.float32),
    mesh=_mesh,
    scratch_types=[
        pltpu.VMEM((NCH, CH), jnp.int32),
        pltpu.VMEM((NCH, CH), jnp.int32),
        pltpu.VMEM((1, CH, D), jnp.float32),
        pltpu.VMEM_SHARED((NPAD, D), jnp.float32),
    ],
    compiler_params=_sc_params,
)
def _prop_kernel(table_hbm, src_hbm, dst_hbm, out_hbm,
                 sidx, didx, rows_v, accum):
    c = lax.axis_index("c")
    s = lax.axis_index("s")
    wid = s * NC + c
    pltpu.sync_copy(src_hbm.at[wid], sidx)
    pltpu.sync_copy(dst_hbm.at[wid], didx)

    zeros16 = jnp.zeros((LANES,), jnp.float32)

    def zbody(i, carry):
        rows_v[0, i // (D // LANES), pl.ds((i % (D // LANES)) * LANES, LANES)] = zeros16
        return carry

    lax.fori_loop(0, CH * (D // LANES), zbody, 0)
    for z in range(ZROWS // CH):
        pltpu.sync_copy(rows_v.at[0], accum.at[pl.ds(s * ZROWS + z * CH, CH)])
    rem = ZROWS % CH
    if rem:
        pltpu.sync_copy(rows_v.at[0].at[pl.ds(0, rem)],
                        accum.at[pl.ds(s * ZROWS + (ZROWS // CH) * CH, rem)])
    plsc.subcore_barrier()

    def ebody(j, carry):
        pltpu.sync_copy(table_hbm.at[sidx.at[j]], rows_v.at[0])
        pltpu.sync_copy(rows_v.at[0], accum.at[didx.at[j]], add=True)
        return carry

    lax.fori_loop(0, NCH, ebody, 0)
    plsc.subcore_barrier()
    pltpu.sync_copy(accum.at[pl.ds(s * ZROWS, ZROWS)],
                    out_hbm.at[c].at[pl.ds(s * ZROWS, ZROWS)])


def _diag_scale(dis2, mat):
    ri = lax.broadcasted_iota(jnp.int32, (D, D), 0)
    ci = lax.broadcasted_iota(jnp.int32, (D, D), 1)
    diag = jnp.where(ri == ci, jnp.broadcast_to(dis2, (D, D)), 0.0)
    return jnp.dot(diag, mat, preferred_element_type=jnp.float32)


def _scale1_body(x_ref, pd_ref, w_ref, h_ref, dis_ref):
    pd = pd_ref[...][0]
    deg = jnp.sum(pd, axis=0, keepdims=True)
    dis = jnp.where(deg > 0.0, lax.rsqrt(jnp.maximum(deg, 1.0)), 0.0)
    xs = _diag_scale(dis, x_ref[...])
    h_ref[...] = jnp.dot(xs, w_ref[...], preferred_element_type=jnp.float32)
    dis_ref[...] = dis.reshape(1, 1, D)


def _scale2_body(p_ref, dis_ref, w_ref, b_ref, h_ref):
    p = p_ref[...]
    psum = p[0] + p[1]
    dis = dis_ref[...].reshape(1, D)
    h1 = _diag_scale(dis, psum) + b_ref[...]
    h_ref[...] = jnp.dot(_diag_scale(dis, h1), w_ref[...],
                         preferred_element_type=jnp.float32)


def _final_body(p_ref, dis_ref, b_ref, out_ref):
    p = p_ref[...]
    psum = p[0] + p[1]
    dis = dis_ref[...].reshape(1, D)
    out_ref[...] = _diag_scale(dis, psum) + b_ref[...]


BLK = 128
GRID = NPAD // BLK


def kernel(x, edge_index, W1, b1, W2, b2):
    src = edge_index[0]
    dst = edge_index[1]
    pad = EPAD - E
    src_p = jnp.concatenate([src, jnp.zeros((pad,), jnp.int32)])
    dst_p = jnp.concatenate([dst, jnp.full((pad,), DUMMY, jnp.int32)])
    src3 = src_p.reshape(NW, NCH, CH)
    dst3 = dst_p.reshape(NW, NCH, CH)
    dst2 = dst_p.reshape(NW, EPT)
    x_pad = jnp.pad(x, ((0, NPAD - N), (0, 0)))
    b1r = b1.reshape(1, D)
    b2r = b2.reshape(1, D)

    pd = _deg_kernel(dst2)
    pd3 = pd.reshape(NW, GRID, BLK).transpose(1, 0, 2)

    h1s, dis3 = pl.pallas_call(
        _scale1_body,
        grid=(GRID,),
        in_specs=[
            pl.BlockSpec((BLK, D), lambda i: (i, 0)),
            pl.BlockSpec((1, NW, BLK), lambda i: (i, 0, 0)),
            pl.BlockSpec((D, D), lambda i: (0, 0)),
        ],
        out_specs=[
            pl.BlockSpec((BLK, D), lambda i: (i, 0)),
            pl.BlockSpec((1, 1, BLK), lambda i: (i, 0, 0)),
        ],
        out_shape=[
            jax.ShapeDtypeStruct((NPAD, D), jnp.float32),
            jax.ShapeDtypeStruct((GRID, 1, BLK), jnp.float32),
        ],
    )(x_pad, pd3, W1)

    p1 = _prop_kernel(h1s, src3, dst3)

    h2s = pl.pallas_call(
        _scale2_body,
        grid=(GRID,),
        in_specs=[
            pl.BlockSpec((NC, BLK, D), lambda i: (0, i, 0)),
            pl.BlockSpec((1, 1, BLK), lambda i: (i, 0, 0)),
            pl.BlockSpec((D, D), lambda i: (0, 0)),
            pl.BlockSpec((1, D), lambda i: (0, 0)),
        ],
        out_specs=pl.BlockSpec((BLK, D), lambda i: (i, 0)),
        out_shape=jax.ShapeDtypeStruct((NPAD, D), jnp.float32),
    )(p1, dis3, W2, b1r)

    p2 = _prop_kernel(h2s, src3, dst3)

    out = pl.pallas_call(
        _final_body,
        grid=(GRID,),
        in_specs=[
            pl.BlockSpec((NC, BLK, D), lambda i: (0, i, 0)),
            pl.BlockSpec((1, 1, BLK), lambda i: (i, 0, 0)),
            pl.BlockSpec((1, D), lambda i: (0, 0)),
        ],
        out_specs=pl.BlockSpec((BLK, D), lambda i: (i, 0)),
        out_shape=jax.ShapeDtypeStruct((NPAD, D), jnp.float32),
    )(p2, dis3, b2r)

    return out[:N]


# speedup vs baseline: 2.3421x; 2.3022x over previous
import functools

import jax
import jax.numpy as jnp
from jax import lax
from jax.experimental import pallas as pl
from jax.experimental.pallas import tpu as pltpu
from jax.experimental.pallas import tpu_sc as plsc

N = 10000
D = 128
E = 320000

NC = 2
NS = 16
NW = NC * NS
LANES = 16

NPAD = 10112
DUMMY = N
CH = 128
NCH = 80
EPT = NCH * CH
EPAD = NW * EPT
ZROWS = NPAD // NS

_mesh = plsc.VectorSubcoreMesh(
    core_axis_name="c", subcore_axis_name="s", num_cores=NC, num_subcores=NS)
_sc_params = pltpu.CompilerParams(needs_layout_passes=False)


@functools.partial(
    pl.kernel,
    out_type=jax.ShapeDtypeStruct((NW, NPAD), jnp.float32),
    mesh=_mesh,
    scratch_types=[
        pltpu.VMEM((EPT,), jnp.int32),
        pltpu.VMEM((NPAD,), jnp.float32),
    ],
    compiler_params=_sc_params,
)
def _deg_kernel(dst_hbm, out_hbm, dst_v, hist_v):
    c = lax.axis_index("c")
    s = lax.axis_index("s")
    wid = s * NC + c
    pltpu.sync_copy(dst_hbm.at[wid], dst_v)
    zeros16 = jnp.zeros((LANES,), jnp.float32)
    ones16 = jnp.ones((LANES,), jnp.float32)

    def zbody(i, carry):
        hist_v[pl.ds(i * LANES, LANES)] = zeros16
        return carry

    lax.fori_loop(0, NPAD // LANES, zbody, 0)

    def ebody(i, carry):
        idx = dst_v[pl.ds(i * LANES, LANES)]
        plsc.addupdate_scatter(hist_v, [idx], ones16)
        return carry

    lax.fori_loop(0, EPT // LANES, ebody, 0)
    pltpu.sync_copy(hist_v, out_hbm.at[wid])


@functools.partial(
    pl.kernel,
    out_type=jax.ShapeDtypeStruct((NC, NPAD, D), jnp.float32),
    mesh=_mesh,
    scratch_types=[
        pltpu.VMEM((NCH, CH), jnp.int32),
        pltpu.VMEM((NCH, CH), jnp.int32),
        pltpu.VMEM((1, CH, D), jnp.float32),
        pltpu.VMEM_SHARED((NPAD, D), jnp.float32),
    ],
    compiler_params=_sc_params,
)
def _prop_kernel(table_hbm, src_hbm, dst_hbm, out_hbm,
                 sidx, didx, rows_v, accum):
    c = lax.axis_index("c")
    s = lax.axis_index("s")
    wid = s * NC + c
    pltpu.sync_copy(src_hbm.at[wid], sidx)
    pltpu.sync_copy(dst_hbm.at[wid], didx)

    zeros16 = jnp.zeros((LANES,), jnp.float32)

    def zbody(i, carry):
        rows_v[0, i // (D // LANES), pl.ds((i % (D // LANES)) * LANES, LANES)] = zeros16
        return carry

    lax.fori_loop(0, CH * (D // LANES), zbody, 0)
    for z in range(ZROWS // CH):
        pltpu.sync_copy(rows_v.at[0], accum.at[pl.ds(s * ZROWS + z * CH, CH)])
    rem = ZROWS % CH
    if rem:
        pltpu.sync_copy(rows_v.at[0].at[pl.ds(0, rem)],
                        accum.at[pl.ds(s * ZROWS + (ZROWS // CH) * CH, rem)])
    plsc.subcore_barrier()

    def ebody(j, carry):
        pltpu.sync_copy(table_hbm.at[sidx.at[j]], rows_v.at[0])
        pltpu.sync_copy(rows_v.at[0], accum.at[didx.at[j]], add=True)
        return carry

    lax.fori_loop(0, NCH, ebody, 0)
    plsc.subcore_barrier()
    pltpu.sync_copy(accum.at[pl.ds(s * ZROWS, ZROWS)],
                    out_hbm.at[c].at[pl.ds(s * ZROWS, ZROWS)])


def _diag_scale(dis2, mat):
    ri = lax.broadcasted_iota(jnp.int32, (D, D), 0)
    ci = lax.broadcasted_iota(jnp.int32, (D, D), 1)
    diag = jnp.where(ri == ci, jnp.broadcast_to(dis2, (D, D)), 0.0)
    return jnp.dot(diag, mat, preferred_element_type=jnp.float32)


def _scale1_body(x_ref, pd_ref, w_ref, h_ref, dis_ref):
    pd = pd_ref[...][0]
    deg = jnp.sum(pd, axis=0, keepdims=True)
    dis = jnp.where(deg > 0.0, lax.rsqrt(jnp.maximum(deg, 1.0)), 0.0)
    xs = _diag_scale(dis, x_ref[...])
    h_ref[...] = jnp.dot(xs, w_ref[...], preferred_element_type=jnp.float32)
    dis_ref[...] = dis.reshape(1, 1, D)


def _scale2_body(p_ref, dis_ref, w_ref, b_ref, h_ref):
    p = p_ref[...]
    psum = p[0] + p[1]
    dis = dis_ref[...].reshape(1, D)
    h1 = _diag_scale(dis, psum) + b_ref[...]
    h_ref[...] = jnp.dot(_diag_scale(dis, h1), w_ref[...],
                         preferred_element_type=jnp.float32)


def _final_body(p_ref, dis_ref, b_ref, out_ref):
    p = p_ref[...]
    psum = p[0] + p[1]
    dis = dis_ref[...].reshape(1, D)
    out_ref[...] = _diag_scale(dis, psum) + b_ref[...]


BLK = 128
GRID = NPAD // BLK


def kernel(x, edge_index, W1, b1, W2, b2):
    src = edge_index[0]
    dst = edge_index[1]
    ppw = EPT - E // NW
    padv = (N + (jnp.arange(ppw) % (NPAD - N))).astype(jnp.int32)
    pads = jnp.broadcast_to(padv, (NW, ppw))
    srcm = jnp.concatenate([src.reshape(NW, E // NW), pads], axis=1)
    dstm = jnp.concatenate([dst.reshape(NW, E // NW), pads], axis=1)
    src3 = srcm.reshape(NW, NCH, CH)
    dst3 = dstm.reshape(NW, NCH, CH)
    dst2 = dstm.reshape(NW, EPT)
    x_pad = jnp.pad(x, ((0, NPAD - N), (0, 0)))
    b1r = b1.reshape(1, D)
    b2r = b2.reshape(1, D)

    pd = _deg_kernel(dst2)
    pd3 = pd.reshape(NW, GRID, BLK).transpose(1, 0, 2)

    h1s, dis3 = pl.pallas_call(
        _scale1_body,
        grid=(GRID,),
        in_specs=[
            pl.BlockSpec((BLK, D), lambda i: (i, 0)),
            pl.BlockSpec((1, NW, BLK), lambda i: (i, 0, 0)),
            pl.BlockSpec((D, D), lambda i: (0, 0)),
        ],
        out_specs=[
            pl.BlockSpec((BLK, D), lambda i: (i, 0)),
            pl.BlockSpec((1, 1, BLK), lambda i: (i, 0, 0)),
        ],
        out_shape=[
            jax.ShapeDtypeStruct((NPAD, D), jnp.float32),
            jax.ShapeDtypeStruct((GRID, 1, BLK), jnp.float32),
        ],
    )(x_pad, pd3, W1)

    p1 = _prop_kernel(h1s, src3, dst3)

    h2s = pl.pallas_call(
        _scale2_body,
        grid=(GRID,),
        in_specs=[
            pl.BlockSpec((NC, BLK, D), lambda i: (0, i, 0)),
            pl.BlockSpec((1, 1, BLK), lambda i: (i, 0, 0)),
            pl.BlockSpec((D, D), lambda i: (0, 0)),
            pl.BlockSpec((1, D), lambda i: (0, 0)),
        ],
        out_specs=pl.BlockSpec((BLK, D), lambda i: (i, 0)),
        out_shape=jax.ShapeDtypeStruct((NPAD, D), jnp.float32),
    )(p1, dis3, W2, b1r)

    p2 = _prop_kernel(h2s, src3, dst3)

    out = pl.pallas_call(
        _final_body,
        grid=(GRID,),
        in_specs=[
            pl.BlockSpec((NC, BLK, D), lambda i: (0, i, 0)),
            pl.BlockSpec((1, 1, BLK), lambda i: (i, 0, 0)),
            pl.BlockSpec((1, D), lambda i: (0, 0)),
        ],
        out_specs=pl.BlockSpec((BLK, D), lambda i: (i, 0)),
        out_shape=jax.ShapeDtypeStruct((NPAD, D), jnp.float32),
    )(p2, dis3, b2r)

    return out[:N]


# speedup vs baseline: 2.8070x; 1.1985x over previous
import functools

import jax
import jax.numpy as jnp
from jax import lax
from jax.experimental import pallas as pl
from jax.experimental.pallas import tpu as pltpu
from jax.experimental.pallas import tpu_sc as plsc

N = 10000
D = 128
E = 320000

NC = 2
NS = 16
NW = NC * NS
LANES = 16

NPAD = 10112
DUMMY = N
CH = 128
NCH = 80
EPT = NCH * CH
EPAD = NW * EPT
ZROWS = NPAD // NS

_mesh = plsc.VectorSubcoreMesh(
    core_axis_name="c", subcore_axis_name="s", num_cores=NC, num_subcores=NS)
_sc_params = pltpu.CompilerParams(needs_layout_passes=False)


@functools.partial(
    pl.kernel,
    out_type=jax.ShapeDtypeStruct((NW, NPAD), jnp.float32),
    mesh=_mesh,
    scratch_types=[
        pltpu.VMEM((EPT,), jnp.int32),
        pltpu.VMEM((NPAD,), jnp.float32),
    ],
    compiler_params=_sc_params,
)
def _deg_kernel(dst_hbm, out_hbm, dst_v, hist_v):
    c = lax.axis_index("c")
    s = lax.axis_index("s")
    wid = s * NC + c
    pltpu.sync_copy(dst_hbm.at[wid], dst_v)
    zeros16 = jnp.zeros((LANES,), jnp.float32)
    ones16 = jnp.ones((LANES,), jnp.float32)

    def zbody(i, carry):
        hist_v[pl.ds(i * LANES, LANES)] = zeros16
        return carry

    lax.fori_loop(0, NPAD // LANES, zbody, 0)

    def ebody(i, carry):
        idx = dst_v[pl.ds(i * LANES, LANES)]
        plsc.addupdate_scatter(hist_v, [idx], ones16)
        return carry

    lax.fori_loop(0, EPT // LANES, ebody, 0)
    pltpu.sync_copy(hist_v, out_hbm.at[wid])


@functools.partial(
    pl.kernel,
    out_type=jax.ShapeDtypeStruct((NC, NPAD, D), jnp.float32),
    mesh=_mesh,
    scratch_types=[
        pltpu.VMEM((NCH, CH), jnp.int32),
        pltpu.VMEM((2, CH), jnp.int32),
        pltpu.VMEM((2, CH), jnp.int32),
        pltpu.VMEM((2, CH, D), jnp.float32),
        pltpu.VMEM_SHARED((NPAD, D), jnp.float32),
        pltpu.SemaphoreType.DMA,
    ],
    compiler_params=_sc_params,
)
def _prop_kernel(table_hbm, packed_hbm, out_hbm,
                 packed_v, sidx, didx, rows_v, accum, sem):
    c = lax.axis_index("c")
    s = lax.axis_index("s")
    wid = s * NC + c
    pltpu.sync_copy(packed_hbm.at[wid], packed_v)

    def unpack(jn, bn):
        for i in range(CH // LANES):
            v = packed_v[jn, pl.ds(i * LANES, LANES)]
            sidx[bn, pl.ds(i * LANES, LANES)] = v & 0xFFFF
            didx[bn, pl.ds(i * LANES, LANES)] = lax.shift_right_logical(v, 16)

    zeros16 = jnp.zeros((LANES,), jnp.float32)

    def zbody(i, carry):
        rows_v[0, i // (D // LANES), pl.ds((i % (D // LANES)) * LANES, LANES)] = zeros16
        return carry

    lax.fori_loop(0, CH * (D // LANES), zbody, 0)
    for z in range(ZROWS // CH):
        pltpu.sync_copy(rows_v.at[0], accum.at[pl.ds(s * ZROWS + z * CH, CH)])
    rem = ZROWS % CH
    if rem:
        pltpu.sync_copy(rows_v.at[0].at[pl.ds(0, rem)],
                        accum.at[pl.ds(s * ZROWS + (ZROWS // CH) * CH, rem)])
    unpack(0, 0)
    pltpu.async_copy(table_hbm.at[sidx.at[0]], rows_v.at[0], sem)
    plsc.subcore_barrier()

    def ebody(j, carry):
        b = lax.rem(j, 2)

        @pl.when(j + 1 < NCH)
        def _():
            unpack(j + 1, 1 - b)

        pltpu.make_async_copy(table_hbm.at[sidx.at[b]], rows_v.at[b], sem).wait()

        @pl.when(j + 1 < NCH)
        def _():
            pltpu.async_copy(table_hbm.at[sidx.at[1 - b]], rows_v.at[1 - b], sem)

        pltpu.sync_copy(rows_v.at[b], accum.at[didx.at[b]], add=True)
        return carry

    lax.fori_loop(0, NCH, ebody, 0)
    plsc.subcore_barrier()
    pltpu.sync_copy(accum.at[pl.ds(s * ZROWS, ZROWS)],
                    out_hbm.at[c].at[pl.ds(s * ZROWS, ZROWS)])


def _diag_scale(dis2, mat):
    ri = lax.broadcasted_iota(jnp.int32, (D, D), 0)
    ci = lax.broadcasted_iota(jnp.int32, (D, D), 1)
    diag = jnp.where(ri == ci, jnp.broadcast_to(dis2, (D, D)), 0.0)
    return jnp.dot(diag, mat, preferred_element_type=jnp.float32)


def _scale1_body(x_ref, pd_ref, w_ref, h_ref, dis_ref):
    pd = pd_ref[...][0]
    deg = jnp.sum(pd, axis=0, keepdims=True)
    dis = jnp.where(deg > 0.0, lax.rsqrt(jnp.maximum(deg, 1.0)), 0.0)
    xs = _diag_scale(dis, x_ref[...])
    h_ref[...] = jnp.dot(xs, w_ref[...], preferred_element_type=jnp.float32)
    dis_ref[...] = dis.reshape(1, 1, D)


def _scale2_body(p_ref, dis_ref, w_ref, b_ref, h_ref):
    p = p_ref[...]
    psum = p[0] + p[1]
    dis = dis_ref[...].reshape(1, D)
    h1 = _diag_scale(dis, psum) + b_ref[...]
    h_ref[...] = jnp.dot(_diag_scale(dis, h1), w_ref[...],
                         preferred_element_type=jnp.float32)


def _final_body(p_ref, dis_ref, b_ref, out_ref):
    p = p_ref[...]
    psum = p[0] + p[1]
    dis = dis_ref[...].reshape(1, D)
    out_ref[...] = _diag_scale(dis, psum) + b_ref[...]


BLK = 128
GRID = NPAD // BLK


def kernel(x, edge_index, W1, b1, W2, b2):
    src = edge_index[0]
    dst = edge_index[1]
    ppw = EPT - E // NW
    padv = (N + (jnp.arange(ppw) % (NPAD - N))).astype(jnp.int32)
    pads = jnp.broadcast_to(padv, (NW, ppw))
    srcm = jnp.concatenate([src.reshape(NW, E // NW), pads], axis=1)
    dstm = jnp.concatenate([dst.reshape(NW, E // NW), pads], axis=1)
    packed3 = (srcm | (dstm << 16)).reshape(NW, NCH, CH)
    dst2 = dstm.reshape(NW, EPT)
    x_pad = jnp.pad(x, ((0, NPAD - N), (0, 0)))
    b1r = b1.reshape(1, D)
    b2r = b2.reshape(1, D)

    pd = _deg_kernel(dst2)
    pd3 = pd.reshape(NW, GRID, BLK).transpose(1, 0, 2)

    h1s, dis3 = pl.pallas_call(
        _scale1_body,
        grid=(GRID,),
        in_specs=[
            pl.BlockSpec((BLK, D), lambda i: (i, 0)),
            pl.BlockSpec((1, NW, BLK), lambda i: (i, 0, 0)),
            pl.BlockSpec((D, D), lambda i: (0, 0)),
        ],
        out_specs=[
            pl.BlockSpec((BLK, D), lambda i: (i, 0)),
            pl.BlockSpec((1, 1, BLK), lambda i: (i, 0, 0)),
        ],
        out_shape=[
            jax.ShapeDtypeStruct((NPAD, D), jnp.float32),
            jax.ShapeDtypeStruct((GRID, 1, BLK), jnp.float32),
        ],
    )(x_pad, pd3, W1)

    p1 = _prop_kernel(h1s, packed3)

    h2s = pl.pallas_call(
        _scale2_body,
        grid=(GRID,),
        in_specs=[
            pl.BlockSpec((NC, BLK, D), lambda i: (0, i, 0)),
            pl.BlockSpec((1, 1, BLK), lambda i: (i, 0, 0)),
            pl.BlockSpec((D, D), lambda i: (0, 0)),
            pl.BlockSpec((1, D), lambda i: (0, 0)),
        ],
        out_specs=pl.BlockSpec((BLK, D), lambda i: (i, 0)),
        out_shape=jax.ShapeDtypeStruct((NPAD, D), jnp.float32),
    )(p1, dis3, W2, b1r)

    p2 = _prop_kernel(h2s, packed3)

    out = pl.pallas_call(
        _final_body,
        grid=(GRID,),
        in_specs=[
            pl.BlockSpec((NC, BLK, D), lambda i: (0, i, 0)),
            pl.BlockSpec((1, 1, BLK), lambda i: (i, 0, 0)),
            pl.BlockSpec((1, D), lambda i: (0, 0)),
        ],
        out_specs=pl.BlockSpec((BLK, D), lambda i: (i, 0)),
        out_shape=jax.ShapeDtypeStruct((NPAD, D), jnp.float32),
    )(p2, dis3, b2r)

    return out[:N]


# speedup vs baseline: 2.8680x; 1.0218x over previous
import functools

import jax
import jax.numpy as jnp
from jax import lax
from jax.experimental import pallas as pl
from jax.experimental.pallas import tpu as pltpu
from jax.experimental.pallas import tpu_sc as plsc

N = 10000
D = 128
E = 320000

NC = 2
NS = 16
NW = NC * NS
LANES = 16

NPAD = 10112
DUMMY = N
CH = 128
NCH = 80
EPT = NCH * CH
EPAD = NW * EPT
ZROWS = NPAD // NS

_mesh = plsc.VectorSubcoreMesh(
    core_axis_name="c", subcore_axis_name="s", num_cores=NC, num_subcores=NS)
_sc_params = pltpu.CompilerParams(needs_layout_passes=False)


@functools.partial(
    pl.kernel,
    out_type=jax.ShapeDtypeStruct((NW, NPAD), jnp.float32),
    mesh=_mesh,
    scratch_types=[
        pltpu.VMEM((EPT,), jnp.int32),
        pltpu.VMEM((NPAD,), jnp.float32),
    ],
    compiler_params=_sc_params,
)
def _deg_kernel(dst_hbm, out_hbm, dst_v, hist_v):
    c = lax.axis_index("c")
    s = lax.axis_index("s")
    wid = s * NC + c
    pltpu.sync_copy(dst_hbm.at[wid], dst_v)
    zeros16 = jnp.zeros((LANES,), jnp.float32)
    ones16 = jnp.ones((LANES,), jnp.float32)

    def zbody(i, carry):
        hist_v[pl.ds(i * LANES, LANES)] = zeros16
        return carry

    lax.fori_loop(0, NPAD // LANES, zbody, 0)

    def ebody(i, carry):
        idx = dst_v[pl.ds(i * LANES, LANES)]
        plsc.addupdate_scatter(hist_v, [idx], ones16)
        return carry

    lax.fori_loop(0, EPT // LANES, ebody, 0)
    pltpu.sync_copy(hist_v, out_hbm.at[wid])


@functools.partial(
    pl.kernel,
    out_type=jax.ShapeDtypeStruct((NC, NPAD, D), jnp.float32),
    mesh=_mesh,
    scratch_types=[
        pltpu.VMEM((NCH, CH), jnp.int32),
        pltpu.VMEM((2, CH), jnp.int32),
        pltpu.VMEM((2, CH), jnp.int32),
        pltpu.VMEM((2, CH, D), jnp.float32),
        pltpu.VMEM_SHARED((NPAD, D), jnp.float32),
        pltpu.SemaphoreType.DMA,
    ],
    compiler_params=_sc_params,
)
def _prop_kernel(table_hbm, packed_hbm, out_hbm,
                 packed_v, sidx, didx, rows_v, accum, sem):
    c = lax.axis_index("c")
    s = lax.axis_index("s")
    wid = s * NC + c
    pltpu.sync_copy(packed_hbm.at[wid], packed_v)

    def unpack(jn, bn):
        for i in range(CH // LANES):
            v = packed_v[jn, pl.ds(i * LANES, LANES)]
            sidx[bn, pl.ds(i * LANES, LANES)] = v & 0xFFFF
            didx[bn, pl.ds(i * LANES, LANES)] = lax.shift_right_logical(v, 16)

    zeros16 = jnp.zeros((LANES,), jnp.float32)

    def zbody(i, carry):
        rows_v[0, i // (D // LANES), pl.ds((i % (D // LANES)) * LANES, LANES)] = zeros16
        return carry

    lax.fori_loop(0, CH * (D // LANES), zbody, 0)
    for z in range(ZROWS // CH):
        pltpu.sync_copy(rows_v.at[0], accum.at[pl.ds(s * ZROWS + z * CH, CH)])
    rem = ZROWS % CH
    if rem:
        pltpu.sync_copy(rows_v.at[0].at[pl.ds(0, rem)],
                        accum.at[pl.ds(s * ZROWS + (ZROWS // CH) * CH, rem)])
    unpack(0, 0)
    pltpu.async_copy(table_hbm.at[sidx.at[0]], rows_v.at[0], sem)
    plsc.subcore_barrier()

    def ebody(j, carry):
        b = lax.rem(j, 2)

        @pl.when(j + 1 < NCH)
        def _():
            unpack(j + 1, 1 - b)

        pltpu.make_async_copy(table_hbm.at[sidx.at[b]], rows_v.at[b], sem).wait()

        @pl.when(j + 1 < NCH)
        def _():
            pltpu.async_copy(table_hbm.at[sidx.at[1 - b]], rows_v.at[1 - b], sem)

        pltpu.sync_copy(rows_v.at[b], accum.at[didx.at[b]], add=True)
        return carry

    lax.fori_loop(0, NCH, ebody, 0)
    plsc.subcore_barrier()
    pltpu.sync_copy(accum.at[pl.ds(s * ZROWS, ZROWS)],
                    out_hbm.at[c].at[pl.ds(s * ZROWS, ZROWS)])


BLK = 128
GRID = NPAD // BLK


def _dis_body(pd_ref, disT_ref):
    deg = jnp.sum(pd_ref[...], axis=0)
    dis = jnp.where(deg > 0.0, lax.rsqrt(jnp.maximum(deg, 1.0)), 0.0)
    disp = jnp.concatenate([dis, jnp.zeros((GPAD - GRID, BLK), jnp.float32)], 0)
    disT_ref[...] = jnp.transpose(disp)


def _col(disT, i):
    ci = lax.broadcasted_iota(jnp.int32, (BLK, GPAD), 1)
    return jnp.sum(jnp.where(ci == i, disT, 0.0), axis=1, keepdims=True)


def _scale1_body(x_ref, disT_ref, w_ref, h_ref):
    dis = _col(disT_ref[...], pl.program_id(0))
    h_ref[...] = jnp.dot(x_ref[...] * dis, w_ref[...],
                         preferred_element_type=jnp.float32)


def _scale2_body(p_ref, disT_ref, w_ref, b_ref, h_ref):
    p = p_ref[...]
    dis = _col(disT_ref[...], pl.program_id(0))
    h1 = (p[0] + p[1]) * dis + b_ref[...]
    h_ref[...] = jnp.dot(h1 * dis, w_ref[...],
                         preferred_element_type=jnp.float32)


def _final_body(p_ref, disT_ref, b_ref, out_ref):
    p = p_ref[...]
    dis = _col(disT_ref[...], pl.program_id(0))
    out_ref[...] = (p[0] + p[1]) * dis + b_ref[...]


GPAD = 80


def kernel(x, edge_index, W1, b1, W2, b2):
    src = edge_index[0]
    dst = edge_index[1]
    ppw = EPT - E // NW
    padv = (N + (jnp.arange(ppw) % (NPAD - N))).astype(jnp.int32)
    pads = jnp.broadcast_to(padv, (NW, ppw))
    srcm = jnp.concatenate([src.reshape(NW, E // NW), pads], axis=1)
    dstm = jnp.concatenate([dst.reshape(NW, E // NW), pads], axis=1)
    packed3 = (srcm | (dstm << 16)).reshape(NW, NCH, CH)
    dst2 = dstm.reshape(NW, EPT)
    x_pad = jnp.pad(x, ((0, NPAD - N), (0, 0)))
    b1r = b1.reshape(1, D)
    b2r = b2.reshape(1, D)

    pd = _deg_kernel(dst2)
    pd3 = pd.reshape(NW, GRID, BLK)

    disT = pl.pallas_call(
        _dis_body,
        out_shape=jax.ShapeDtypeStruct((BLK, GPAD), jnp.float32),
    )(pd3)

    h1s = pl.pallas_call(
        _scale1_body,
        grid=(GRID,),
        in_specs=[
            pl.BlockSpec((BLK, D), lambda i: (i, 0)),
            pl.BlockSpec((BLK, GPAD), lambda i: (0, 0)),
            pl.BlockSpec((D, D), lambda i: (0, 0)),
        ],
        out_specs=pl.BlockSpec((BLK, D), lambda i: (i, 0)),
        out_shape=jax.ShapeDtypeStruct((NPAD, D), jnp.float32),
    )(x_pad, disT, W1)

    p1 = _prop_kernel(h1s, packed3)

    h2s = pl.pallas_call(
        _scale2_body,
        grid=(GRID,),
        in_specs=[
            pl.BlockSpec((NC, BLK, D), lambda i: (0, i, 0)),
            pl.BlockSpec((BLK, GPAD), lambda i: (0, 0)),
            pl.BlockSpec((D, D), lambda i: (0, 0)),
            pl.BlockSpec((1, D), lambda i: (0, 0)),
        ],
        out_specs=pl.BlockSpec((BLK, D), lambda i: (i, 0)),
        out_shape=jax.ShapeDtypeStruct((NPAD, D), jnp.float32),
    )(p1, disT, W2, b1r)

    p2 = _prop_kernel(h2s, packed3)

    out = pl.pallas_call(
        _final_body,
        grid=(GRID,),
        in_specs=[
            pl.BlockSpec((NC, BLK, D), lambda i: (0, i, 0)),
            pl.BlockSpec((BLK, GPAD), lambda i: (0, 0)),
            pl.BlockSpec((1, D), lambda i: (0, 0)),
        ],
        out_specs=pl.BlockSpec((BLK, D), lambda i: (i, 0)),
        out_shape=jax.ShapeDtypeStruct((NPAD, D), jnp.float32),
    )(p2, disT, b2r)

    return out[:N]
